# baseline (speedup 1.0000x reference)
import functools

import jax
import jax.numpy as jnp
from jax import lax
from jax.experimental import pallas as pl
from jax.experimental.pallas import tpu as pltpu
from jax.experimental.pallas import tpu_sc as plsc

_LANES = 128


@functools.lru_cache(maxsize=None)
def _make_sc_gather(B):
    info = plsc.get_sparse_core_info()
    NC, NS = info.num_cores, info.num_subcores
    NW = NC * NS
    assert B % (8 * NW) == 0
    b_per_w = B // NW
    mesh = plsc.VectorSubcoreMesh(core_axis_name="c", subcore_axis_name="s")

    @functools.partial(
        pl.kernel,
        mesh=mesh,
        compiler_params=pltpu.CompilerParams(use_tc_tiling_on_sc=False),
        out_type=jax.ShapeDtypeStruct((B, _LANES), jnp.float32),
        scratch_types=[
            pltpu.VMEM((b_per_w,), jnp.int32),
            pltpu.VMEM((b_per_w, 64), jnp.float32),
            pltpu.SemaphoreType.DMA,
        ],
    )
    def sc_gather(idx_hbm, tab_hbm, out_hbm, idx_v, rows_v, sem):
        wid = lax.axis_index("s") * NC + lax.axis_index("c")
        base = wid * b_per_w
        pltpu.sync_copy(idx_hbm.at[pl.ds(base, b_per_w)], idx_v)
        pltpu.async_copy(tab_hbm.at[idx_v], rows_v, sem).wait()
        pltpu.sync_copy(rows_v, out_hbm.at[pl.ds(base, b_per_w), pl.ds(0, 64)])

    return sc_gather


def _outer_sum_body(S, D, y_ref, x_ref, o_ref, xt_scr):
    i = pl.program_id(0)

    @pl.when(i == 0)
    def _prologue():
        for j in range(S):
            xt_scr[j] = jnp.swapaxes(x_ref[j][:, :D], 0, 1)

    yt = jnp.swapaxes(y_ref[0][:, :D], 0, 1)
    for j in range(S):
        o_ref[0, j] = yt + xt_scr[j]


@functools.lru_cache(maxsize=None)
def _make_outer_sum(N, S, D):
    return pl.pallas_call(
        functools.partial(_outer_sum_body, S, D),
        grid=(S,),
        in_specs=[
            pl.BlockSpec((1, N, _LANES), lambda i: (i, 0, 0)),
            pl.BlockSpec((S, N, _LANES), lambda i: (0, 0, 0)),
        ],
        out_specs=pl.BlockSpec((1, S, D, N), lambda i: (i, 0, 0, 0)),
        out_shape=jax.ShapeDtypeStruct((S, S, D, N), jnp.float32),
        scratch_shapes=[pltpu.VMEM((S, D, N), jnp.float32)],
    )


def kernel(y_indexes, x_indexes, x_table, y_table):
    N, S = x_indexes.shape
    D = x_table.shape[1]
    B = N * S

    yi = y_indexes.T.reshape(B).astype(jnp.int32)
    xi = x_indexes.T.reshape(B).astype(jnp.int32)

    yt_l = jax.lax.optimization_barrier(
        y_table.reshape(-1, _LANES)).reshape(y_table.shape)
    xt_l = jax.lax.optimization_barrier(
        x_table.reshape(-1, _LANES)).reshape(x_table.shape)

    gather = _make_sc_gather(B)
    y2 = gather(yi, yt_l)
    x2 = gather(xi, xt_l)

    out_phys = _make_outer_sum(N, S, D)(
        y2.reshape(S, N, _LANES), x2.reshape(S, N, _LANES))
    return jnp.transpose(out_phys, (3, 2, 0, 1))
